# baseline (speedup 1.0000x reference)
import jax
import jax.numpy as jnp
from jax import lax
from jax.experimental import pallas as pl
from jax.experimental.pallas import tpu as pltpu
from jax.experimental.pallas import tpu_sc as plsc

K = 64
N_ROWS = 128
N_COLS = 32768
NC = 2
NS = 16
NW = NC * NS
ROWS_PER_W = N_ROWS // NW
VREGS = N_COLS // 16
WINDOWS = VREGS // 16
SURV = N_COLS


def _sortd(v):
    s, _ = plsc.sort_key_val(v, v, descending=True)
    return s


def _rev(v):
    return lax.rev(v, (0,))


def _merge2(a, b):
    br = _rev(b)
    hi = jnp.maximum(a, br)
    lo = jnp.minimum(a, br)
    return _sortd(hi), _sortd(lo)


def _merge32(a0, a1, b0, b1):
    rb0, rb1 = _rev(b1), _rev(b0)
    hi0 = jnp.maximum(a0, rb0)
    hi1 = jnp.maximum(a1, rb1)
    lo0 = jnp.minimum(a0, rb0)
    lo1 = jnp.minimum(a1, rb1)
    h0 = jnp.maximum(hi0, hi1)
    h1 = jnp.minimum(hi0, hi1)
    l0 = jnp.maximum(lo0, lo1)
    l1 = jnp.minimum(lo0, lo1)
    return _sortd(h0), _sortd(h1), _sortd(l0), _sortd(l1)


def _sort64(c0, c1, c2, c3):
    a0, a1 = _merge2(_sortd(c0), _sortd(c1))
    b0, b1 = _merge2(_sortd(c2), _sortd(c3))
    return _merge32(a0, a1, b0, b1)


def _merge_top64(t, c):
    t0, t1, t2, t3 = t
    c0, c1, c2, c3 = c
    h0 = jnp.maximum(t0, _rev(c3))
    h1 = jnp.maximum(t1, _rev(c2))
    h2 = jnp.maximum(t2, _rev(c1))
    h3 = jnp.maximum(t3, _rev(c0))
    p0 = jnp.maximum(h0, h2)
    p2 = jnp.minimum(h0, h2)
    p1 = jnp.maximum(h1, h3)
    p3 = jnp.minimum(h1, h3)
    q0 = jnp.maximum(p0, p1)
    q1 = jnp.minimum(p0, p1)
    q2 = jnp.maximum(p2, p3)
    q3 = jnp.minimum(p2, p3)
    return _sortd(q0), _sortd(q1), _sortd(q2), _sortd(q3)


def _row_top64(row_v, surv_v, colmax_v, gsel_v):
    neg = jnp.full((16,), -jnp.inf, jnp.float32)
    iota = jax.lax.iota(jnp.int32, 16)

    def p1_body(w, r):
        r0, r1, r2, r3 = r
        base = w * 256
        m = row_v[pl.ds(base, 16)]
        for t in range(1, 16):
            m = jnp.maximum(m, row_v[pl.ds(base + 16 * t, 16)])
        colmax_v[pl.ds(w * 16, 16)] = m
        n0 = jnp.maximum(r0, m)
        x = jnp.minimum(r0, m)
        n1 = jnp.maximum(r1, x)
        x = jnp.minimum(r1, x)
        n2 = jnp.maximum(r2, x)
        x = jnp.minimum(r2, x)
        n3 = jnp.maximum(r3, x)
        return n0, n1, n2, n3

    r = lax.fori_loop(0, WINDOWS, p1_body, (neg, neg, neg, neg))
    tau = jnp.full((16,), jnp.min(r[3]), jnp.float32)

    def sel_body(j, selcnt):
        for u in range(4):
            w = j * 4 + u
            m = colmax_v[pl.ds(w * 16, 16)]
            mask = m >= tau
            wv = jnp.full((16,), w, jnp.int32)
            plsc.store_scatter(gsel_v, [selcnt + iota], wv, mask=mask)
            selcnt = selcnt + jnp.where(mask, 16, 0)
        return selcnt

    selcnt = lax.fori_loop(0, WINDOWS // 4, sel_body,
                           jnp.zeros((16,), jnp.int32))
    maxsel = jnp.max(selcnt)

    def p2_body(d, cnt16):
        wvec = gsel_v[pl.ds(d * 16, 16)]
        valid = (d * 16) < selcnt
        base = jnp.where(valid, wvec, 0) * 256 + iota
        for t in range(16):
            v = plsc.load_gather(row_v, [base + t * 16])
            mask = valid & (v >= tau)
            plsc.store_scatter(surv_v, [cnt16 + iota], v, mask=mask)
            cnt16 = cnt16 + jnp.where(mask, 16, 0)
        return cnt16

    cnt16 = lax.fori_loop(0, (maxsel + 15) // 16, p2_body,
                          jnp.zeros((16,), jnp.int32))

    maxc = jnp.max(cnt16)
    nblk = (maxc + 48) // 64

    def p3_body(c, t):
        vs = []
        for u in range(4):
            d = c * 4 + u
            v = surv_v[pl.ds(d * 16, 16)]
            vs.append(jnp.where(d * 16 < cnt16, v, neg))
        return _merge_top64(t, _sort64(vs[0], vs[1], vs[2], vs[3]))

    return lax.fori_loop(0, nblk, p3_body, (neg, neg, neg, neg))


def _tec_body(x_hbm, out_hbm, row_a, row_b, surv_v, colmax_v, gsel_v,
              out_v, sem_a, sem_b):
    wid = lax.axis_index("s") * NC + lax.axis_index("c")
    row0 = wid * ROWS_PER_W
    bufs = (row_a, row_b)
    sems = (sem_a, sem_b)
    copies = [pltpu.async_copy(x_hbm.at[row0], row_a, sem_a)]
    for i in range(ROWS_PER_W):
        if i + 1 < ROWS_PER_W:
            copies.append(pltpu.async_copy(
                x_hbm.at[row0 + i + 1], bufs[(i + 1) % 2], sems[(i + 1) % 2]))
        copies[i].wait()
        t = _row_top64(bufs[i % 2], surv_v, colmax_v, gsel_v)
        for k in range(4):
            out_v[i, pl.ds(16 * k, 16)] = t[k]
    pltpu.sync_copy(out_v, out_hbm.at[pl.ds(row0, ROWS_PER_W)])


def kernel(x):
    mesh = plsc.VectorSubcoreMesh(core_axis_name="c", subcore_axis_name="s")
    run = pl.kernel(
        _tec_body,
        mesh=mesh,
        out_type=jax.ShapeDtypeStruct((N_ROWS, K), jnp.float32),
        scratch_types=[
            pltpu.VMEM((N_COLS,), jnp.float32),
            pltpu.VMEM((N_COLS,), jnp.float32),
            pltpu.VMEM((SURV,), jnp.float32),
            pltpu.VMEM((VREGS,), jnp.float32),
            pltpu.VMEM((VREGS,), jnp.int32),
            pltpu.VMEM((ROWS_PER_W, K), jnp.float32),
            pltpu.SemaphoreType.DMA,
            pltpu.SemaphoreType.DMA,
        ],
        compiler_params=pltpu.CompilerParams(needs_layout_passes=False),
    )
    return run(x)
